# baseline (speedup 1.0000x reference)
import functools

import jax
import jax.numpy as jnp
from jax import lax
from jax.experimental import pallas as pl
from jax.experimental.pallas import tpu as pltpu
from jax.experimental.pallas import tpu_sc as plsc

SIGMA_MIN = 0.02
SIGMA_MAX = 0.3
EPS = 1e-09

NC = 2
NS = 16
L = 16
NW = NC * NS
LW = 128
SLOT = 64


def _encoder_call(rel_id, tau, T_t, B, DIM, K):
    BPW = B // NW
    C = 128
    NCHUNK = BPW // C
    HG = DIM // L
    AF = K * DIM
    TW = 384
    ORC = C * DIM // LW

    mesh = plsc.VectorSubcoreMesh(
        core_axis_name="c", subcore_axis_name="s",
        num_cores=NC, num_subcores=NS)

    f32 = jnp.float32

    @functools.partial(
        pl.kernel,
        out_type=(
            jax.ShapeDtypeStruct((B * DIM // LW, LW), f32),
            jax.ShapeDtypeStruct((B * DIM // LW, LW), f32),
            jax.ShapeDtypeStruct((B * DIM // LW, LW), f32),
        ),
        mesh=mesh,
        compiler_params=pltpu.CompilerParams(use_tc_tiling_on_sc=True),
        scratch_types=[
            pltpu.VMEM((BPW,), jnp.int32),
            pltpu.VMEM((BPW,), f32),
            pltpu.VMEM((C, TW), f32), pltpu.VMEM((C, TW), f32),
            pltpu.VMEM((ORC, LW), f32), pltpu.VMEM((ORC, LW), f32),
            pltpu.VMEM((ORC, LW), f32), pltpu.VMEM((ORC, LW), f32),
            pltpu.VMEM((ORC, LW), f32), pltpu.VMEM((ORC, LW), f32),
            pltpu.SemaphoreType.DMA,
            pltpu.SemaphoreType.DMA,
            pltpu.SemaphoreType.DMA,
            pltpu.SemaphoreType.DMA,
        ],
    )
    def enc(rel_hbm, tau_hbm, t_hbm,
            de_hbm, dt_hbm, dp_hbm,
            idxv, tv,
            av0, av1,
            oe0, oe1, ot0, ot1, op0, op1,
            semg0, semg1, semo0, semo1):
        wid = lax.axis_index("s") * NC + lax.axis_index("c")
        base = pl.multiple_of(wid * BPW, BPW)
        gbufs = ((av0, semg0), (av1, semg1))
        obufs = ((oe0, ot0, op0, semo0), (oe1, ot1, op1, semo1))

        pltpu.sync_copy(rel_hbm.at[pl.ds(base, BPW)], idxv)
        pltpu.sync_copy(tau_hbm.at[pl.ds(base, BPW)], tv)

        shuf = jnp.full((L,), 8, jnp.int32) + (lax.iota(jnp.int32, L) & 7)
        gdn = lax.GatherDimensionNumbers(
            offset_dims=(), collapsed_slice_dims=(0,),
            start_index_map=(0,))

        def start(c, slot):
            av, sem = gbufs[slot]
            return (
                pltpu.async_copy(
                    t_hbm.at[idxv.at[pl.ds(c * C, C)]], av, sem),
            )

        def compute(c, slot, oslot):
            av, _ = gbufs[slot]
            oe, ot, op, _ = obufs[oslot]

            def obody(g, carry):
                b0 = c * C + g * L
                tvec = tv[pl.ds(b0, L)]
                for i in range(L):
                    b = g * L + i
                    tau_b = tvec[i]
                    row = av[b, pl.ds(AF + DIM, L)]
                    sig = SIGMA_MIN + (SIGMA_MAX - SIGMA_MIN) / (
                        1.0 + jnp.exp(-row))
                    den = lax.gather(
                        2.0 * sig * sig + EPS, shuf[:, None], gdn, (1,),
                        mode=lax.GatherScatterMode.PROMISE_IN_BOUNDS)
                    d = tau_b - row
                    gvec = jnp.exp(-(d * d) / den)
                    for h in range(HG):
                        fo = i * DIM + h * L
                        orow = g * (L * DIM // LW) + fo // LW
                        ocol = fo % LW
                        w16 = av[b, pl.ds(AF + h * L, L)]
                        dth = w16 * tau_b
                        acc0 = av[b, pl.ds(h * L, L)] * gvec[0]
                        acc1 = av[b, pl.ds(DIM + h * L, L)] * gvec[1]
                        for k in range(2, K):
                            a16 = av[b, pl.ds(k * DIM + h * L, L)]
                            if k % 2 == 0:
                                acc0 = acc0 + a16 * gvec[k]
                            else:
                                acc1 = acc1 + a16 * gvec[k]
                        acc = acc0 + acc1
                        ot[orow, pl.ds(ocol, L)] = dth
                        op[orow, pl.ds(ocol, L)] = acc
                        oe[orow, pl.ds(ocol, L)] = dth + acc
                return carry

            lax.fori_loop(0, C // L, obody, 0)

        def flush(c, oslot):
            oe, ot, op, sem = obufs[oslot]
            oro = pl.multiple_of((base + c * C) * DIM // LW, ORC)
            return (
                pltpu.async_copy(oe, de_hbm.at[pl.ds(oro, ORC)], sem),
                pltpu.async_copy(ot, dt_hbm.at[pl.ds(oro, ORC)], sem),
                pltpu.async_copy(op, dp_hbm.at[pl.ds(oro, ORC)], sem),
            )

        pend = start(0, 0)
        oflush = [None, None]
        for c in range(NCHUNK):
            slot = c & 1
            cur = pend
            if c + 1 < NCHUNK:
                pend = start(c + 1, 1 - slot)
            for cp in cur:
                cp.wait()
            if oflush[slot] is not None:
                for cp in oflush[slot]:
                    cp.wait()
            compute(c, slot, slot)
            oflush[slot] = flush(c, slot)
        for fl in oflush:
            if fl is not None:
                for cp in fl:
                    cp.wait()

    return enc(rel_id, tau, T_t)


def kernel(rel_id, tau, W_trend, A, mu, s):
    N, K, DIM = A.shape
    B = rel_id.shape[0]
    T_t = jnp.concatenate(
        [A.reshape(N, K * DIM), W_trend, mu, s,
         jnp.zeros((N, 80), jnp.float32)],
        axis=1)
    de, dt, dp = _encoder_call(rel_id.astype(jnp.int32), tau,
                               T_t, B, DIM, K)
    return (de.reshape(B, DIM), dt.reshape(B, DIM), dp.reshape(B, DIM))


# speedup vs baseline: 1.4884x; 1.4884x over previous
import functools

import jax
import jax.numpy as jnp
from jax import lax
from jax.experimental import pallas as pl
from jax.experimental.pallas import tpu as pltpu
from jax.experimental.pallas import tpu_sc as plsc

SIGMA_MIN = 0.02
SIGMA_MAX = 0.3
EPS = 1e-09

NC = 2
NS = 16
L = 16
NW = NC * NS
LW = 128
SLOT = 64


def _encoder_call(rel_id, tau, st_t, A_t, B, DIM, K):
    BPW = B // NW
    C = 128
    NCHUNK = BPW // C
    HG = DIM // L
    AF = K * DIM
    ORC = C * DIM // LW

    mesh = plsc.VectorSubcoreMesh(
        core_axis_name="c", subcore_axis_name="s",
        num_cores=NC, num_subcores=NS)

    f32 = jnp.float32

    @functools.partial(
        pl.kernel,
        out_type=(
            jax.ShapeDtypeStruct((B * DIM // LW, LW), f32),
            jax.ShapeDtypeStruct((B * DIM // LW, LW), f32),
            jax.ShapeDtypeStruct((B * DIM // LW, LW), f32),
        ),
        mesh=mesh,
        compiler_params=pltpu.CompilerParams(use_tc_tiling_on_sc=True),
        scratch_types=[
            pltpu.VMEM((BPW,), jnp.int32),
            pltpu.VMEM((BPW,), f32),
            pltpu.VMEM((C, LW), f32), pltpu.VMEM((C, LW), f32),
            pltpu.VMEM((C, AF), f32), pltpu.VMEM((C, AF), f32),
            pltpu.VMEM((ORC, LW), f32), pltpu.VMEM((ORC, LW), f32),
            pltpu.VMEM((ORC, LW), f32), pltpu.VMEM((ORC, LW), f32),
            pltpu.VMEM((ORC, LW), f32), pltpu.VMEM((ORC, LW), f32),
            pltpu.SemaphoreType.DMA,
            pltpu.SemaphoreType.DMA,
            pltpu.SemaphoreType.DMA,
            pltpu.SemaphoreType.DMA,
        ],
    )
    def enc(rel_hbm, tau_hbm, st_hbm, a_hbm,
            de_hbm, dt_hbm, dp_hbm,
            idxv, tv,
            sv0, sv1, av0, av1,
            oe0, oe1, ot0, ot1, op0, op1,
            semg0, semg1, semo0, semo1):
        wid = lax.axis_index("s") * NC + lax.axis_index("c")
        base = pl.multiple_of(wid * BPW, BPW)
        gbufs = ((sv0, av0, semg0), (sv1, av1, semg1))
        obufs = ((oe0, ot0, op0, semo0), (oe1, ot1, op1, semo1))

        pltpu.sync_copy(rel_hbm.at[pl.ds(base, BPW)], idxv)
        pltpu.sync_copy(tau_hbm.at[pl.ds(base, BPW)], tv)

        shuf = jnp.full((L,), 8, jnp.int32) + (lax.iota(jnp.int32, L) & 7)
        gdn = lax.GatherDimensionNumbers(
            offset_dims=(), collapsed_slice_dims=(0,),
            start_index_map=(0,))

        def start(c, slot):
            sv, av, sem = gbufs[slot]
            return (
                pltpu.async_copy(
                    st_hbm.at[idxv.at[pl.ds(c * C, C)]], sv, sem),
                pltpu.async_copy(
                    a_hbm.at[idxv.at[pl.ds(c * C, C)]], av, sem),
            )

        def compute(c, slot, oslot):
            sv, av, _ = gbufs[slot]
            oe, ot, op, _ = obufs[oslot]

            def obody(g, carry):
                b0 = c * C + g * L
                tvec = tv[pl.ds(b0, L)]
                for i in range(L):
                    b = g * L + i
                    tau_b = tvec[i]
                    slot_o = 0
                    row = sv[b, pl.ds(slot_o + DIM, L)]
                    sig = SIGMA_MIN + (SIGMA_MAX - SIGMA_MIN) / (
                        1.0 + jnp.exp(-row))
                    den = lax.gather(
                        2.0 * sig * sig + EPS, shuf[:, None], gdn, (1,),
                        mode=lax.GatherScatterMode.PROMISE_IN_BOUNDS)
                    d = tau_b - row
                    gvec = jnp.exp(-(d * d) / den)
                    for h in range(HG):
                        fo = i * DIM + h * L
                        orow = g * (L * DIM // LW) + fo // LW
                        ocol = fo % LW
                        w16 = sv[b, pl.ds(slot_o + h * L, L)]
                        dth = w16 * tau_b
                        acc0 = av[b, pl.ds(h * L, L)] * gvec[0]
                        acc1 = av[b, pl.ds(DIM + h * L, L)] * gvec[1]
                        for k in range(2, K):
                            a16 = av[b, pl.ds(k * DIM + h * L, L)]
                            if k % 2 == 0:
                                acc0 = acc0 + a16 * gvec[k]
                            else:
                                acc1 = acc1 + a16 * gvec[k]
                        acc = acc0 + acc1
                        ot[orow, pl.ds(ocol, L)] = dth
                        op[orow, pl.ds(ocol, L)] = acc
                        oe[orow, pl.ds(ocol, L)] = dth + acc
                return carry

            lax.fori_loop(0, C // L, obody, 0)

        def flush(c, oslot):
            oe, ot, op, sem = obufs[oslot]
            oro = pl.multiple_of((base + c * C) * DIM // LW, ORC)
            return (
                pltpu.async_copy(oe, de_hbm.at[pl.ds(oro, ORC)], sem),
                pltpu.async_copy(ot, dt_hbm.at[pl.ds(oro, ORC)], sem),
                pltpu.async_copy(op, dp_hbm.at[pl.ds(oro, ORC)], sem),
            )

        pend = start(0, 0)
        oflush = [None, None]
        for c in range(NCHUNK):
            slot = c & 1
            cur = pend
            if c + 1 < NCHUNK:
                pend = start(c + 1, 1 - slot)
            for cp in cur:
                cp.wait()
            if oflush[slot] is not None:
                for cp in oflush[slot]:
                    cp.wait()
            compute(c, slot, slot)
            oflush[slot] = flush(c, slot)
        for fl in oflush:
            if fl is not None:
                for cp in fl:
                    cp.wait()

    return enc(rel_id, tau, st_t, A_t)


def kernel(rel_id, tau, W_trend, A, mu, s):
    N, K, DIM = A.shape
    B = rel_id.shape[0]
    A_t = A.reshape(N, K * DIM)
    st_t = jnp.concatenate(
        [W_trend, mu, s, jnp.zeros((N, LW - DIM - 2 * K), jnp.float32)],
        axis=1)
    de, dt, dp = _encoder_call(rel_id.astype(jnp.int32), tau,
                               st_t, A_t, B, DIM, K)
    return (de.reshape(B, DIM), dt.reshape(B, DIM), dp.reshape(B, DIM))


# speedup vs baseline: 1.9843x; 1.3332x over previous
import functools

import jax
import jax.numpy as jnp
from jax import lax
from jax.experimental import pallas as pl
from jax.experimental.pallas import tpu as pltpu
from jax.experimental.pallas import tpu_sc as plsc

SIGMA_MIN = 0.02
SIGMA_MAX = 0.3
EPS = 1e-09

NC = 2
NS = 16
L = 16
NW = NC * NS
LW = 128
SLOT = 64


def _trend_and_gauss_call(rel_id, tau, st_t, B, DIM, K):
    BPW = B // NW
    C = 128
    NCHUNK = BPW // C
    HG = DIM // L
    ORC = C * DIM // LW
    GRC = C * L // LW

    mesh = plsc.VectorSubcoreMesh(
        core_axis_name="c", subcore_axis_name="s",
        num_cores=NC, num_subcores=NS)
    f32 = jnp.float32

    @functools.partial(
        pl.kernel,
        out_type=(
            jax.ShapeDtypeStruct((B * DIM // LW, LW), f32),
            jax.ShapeDtypeStruct((B * L // LW, LW), f32),
        ),
        mesh=mesh,
        compiler_params=pltpu.CompilerParams(use_tc_tiling_on_sc=True),
        scratch_types=[
            pltpu.VMEM((BPW,), jnp.int32),
            pltpu.VMEM((BPW,), f32),
            pltpu.VMEM((C, LW), f32), pltpu.VMEM((C, LW), f32),
            pltpu.VMEM((ORC, LW), f32), pltpu.VMEM((ORC, LW), f32),
            pltpu.VMEM((GRC, LW), f32), pltpu.VMEM((GRC, LW), f32),
            pltpu.SemaphoreType.DMA, pltpu.SemaphoreType.DMA,
            pltpu.SemaphoreType.DMA, pltpu.SemaphoreType.DMA,
        ],
    )
    def enc1(rel_hbm, tau_hbm, st_hbm, dt_hbm, gw_hbm,
             idxv, tv, sv0, sv1, ot0, ot1, gs0, gs1,
             semg0, semg1, semo0, semo1):
        wid = lax.axis_index("s") * NC + lax.axis_index("c")
        base = pl.multiple_of(wid * BPW, BPW)
        gbufs = ((sv0, semg0), (sv1, semg1))
        obufs = ((ot0, gs0, semo0), (ot1, gs1, semo1))

        pltpu.sync_copy(rel_hbm.at[pl.ds(base, BPW)], idxv)
        pltpu.sync_copy(tau_hbm.at[pl.ds(base, BPW)], tv)

        shuf = jnp.full((L,), 8, jnp.int32) + (lax.iota(jnp.int32, L) & 7)
        gdn = lax.GatherDimensionNumbers(
            offset_dims=(), collapsed_slice_dims=(0,),
            start_index_map=(0,))

        def start(c, slot):
            sv, sem = gbufs[slot]
            return (pltpu.async_copy(
                st_hbm.at[idxv.at[pl.ds(c * C, C)]], sv, sem),)

        def compute(c, slot):
            sv, _ = gbufs[slot]
            ot, gs, _ = obufs[slot]

            def obody(g, carry):
                b0 = c * C + g * L
                tvec = tv[pl.ds(b0, L)]
                rvec = idxv[pl.ds(b0, L)]
                for i in range(L):
                    b = g * L + i
                    tau_b = tvec[i]
                    slot_o = (rvec[i] & 1) * SLOT
                    row = sv[b, pl.ds(slot_o + DIM, L)]
                    sig = SIGMA_MIN + (SIGMA_MAX - SIGMA_MIN) / (
                        1.0 + jnp.exp(-row))
                    den = lax.gather(
                        2.0 * sig * sig + EPS, shuf[:, None], gdn, (1,),
                        mode=lax.GatherScatterMode.PROMISE_IN_BOUNDS)
                    d = tau_b - row
                    gs[g * 2 + i // 8, pl.ds((i % 8) * L, L)] = jnp.exp(
                        -(d * d) / den)
                    for h in range(HG):
                        fo = i * DIM + h * L
                        w16 = sv[b, pl.ds(slot_o + h * L, L)]
                        ot[g * (L * DIM // LW) + fo // LW,
                           pl.ds(fo % LW, L)] = w16 * tau_b
                return carry

            lax.fori_loop(0, C // L, obody, 0)

        def flush(c, slot):
            ot, gs, sem = obufs[slot]
            oro = pl.multiple_of((base + c * C) * DIM // LW, ORC)
            gro = pl.multiple_of((base + c * C) * L // LW, GRC)
            return (
                pltpu.async_copy(ot, dt_hbm.at[pl.ds(oro, ORC)], sem),
                pltpu.async_copy(gs, gw_hbm.at[pl.ds(gro, GRC)], sem),
            )

        pend = start(0, 0)
        oflush = [None, None]
        for c in range(NCHUNK):
            slot = c & 1
            cur = pend
            if c + 1 < NCHUNK:
                pend = start(c + 1, 1 - slot)
            for cp in cur:
                cp.wait()
            if oflush[slot] is not None:
                for cp in oflush[slot]:
                    cp.wait()
            compute(c, slot)
            oflush[slot] = flush(c, slot)
        for fl in oflush:
            if fl is not None:
                for cp in fl:
                    cp.wait()

    return enc1(rel_id, tau, st_t)


def _pulse_call(rel_id, A_t, dt_t, gw_t, B, DIM, K):
    BPW = B // NW
    C = 128
    NCHUNK = BPW // C
    HG = DIM // L
    AF = K * DIM
    ORC = C * DIM // LW
    GRC = C * L // LW

    mesh = plsc.VectorSubcoreMesh(
        core_axis_name="c", subcore_axis_name="s",
        num_cores=NC, num_subcores=NS)
    f32 = jnp.float32

    @functools.partial(
        pl.kernel,
        out_type=(
            jax.ShapeDtypeStruct((B * DIM // LW, LW), f32),
            jax.ShapeDtypeStruct((B * DIM // LW, LW), f32),
        ),
        mesh=mesh,
        compiler_params=pltpu.CompilerParams(use_tc_tiling_on_sc=True),
        scratch_types=[
            pltpu.VMEM((BPW,), jnp.int32),
            pltpu.VMEM((C, AF), f32), pltpu.VMEM((C, AF), f32),
            pltpu.VMEM((ORC, LW), f32), pltpu.VMEM((ORC, LW), f32),
            pltpu.VMEM((GRC, LW), f32), pltpu.VMEM((GRC, LW), f32),
            pltpu.VMEM((ORC, LW), f32), pltpu.VMEM((ORC, LW), f32),
            pltpu.VMEM((ORC, LW), f32), pltpu.VMEM((ORC, LW), f32),
            pltpu.SemaphoreType.DMA, pltpu.SemaphoreType.DMA,
            pltpu.SemaphoreType.DMA, pltpu.SemaphoreType.DMA,
        ],
    )
    def enc2(rel_hbm, a_hbm, dtin_hbm, gw_hbm, de_hbm, dp_hbm,
             idxv, av0, av1, dv0, dv1, gs0, gs1,
             oe0, oe1, op0, op1,
             semg0, semg1, semo0, semo1):
        wid = lax.axis_index("s") * NC + lax.axis_index("c")
        base = pl.multiple_of(wid * BPW, BPW)
        gbufs = ((av0, dv0, gs0, semg0), (av1, dv1, gs1, semg1))
        obufs = ((oe0, op0, semo0), (oe1, op1, semo1))

        pltpu.sync_copy(rel_hbm.at[pl.ds(base, BPW)], idxv)

        def start(c, slot):
            av, dv, gs, sem = gbufs[slot]
            oro = pl.multiple_of((base + c * C) * DIM // LW, ORC)
            gro = pl.multiple_of((base + c * C) * L // LW, GRC)
            return (
                pltpu.async_copy(
                    a_hbm.at[idxv.at[pl.ds(c * C, C)]], av, sem),
                pltpu.async_copy(dtin_hbm.at[pl.ds(oro, ORC)], dv, sem),
                pltpu.async_copy(gw_hbm.at[pl.ds(gro, GRC)], gs, sem),
            )

        def compute(c, slot):
            av, dv, gs, _ = gbufs[slot]
            oe, op, _ = obufs[slot]

            def obody(g, carry):
                for i in range(L):
                    b = g * L + i
                    gvec = gs[g * 2 + i // 8, pl.ds((i % 8) * L, L)]
                    for h in range(HG):
                        fo = i * DIM + h * L
                        orow = g * (L * DIM // LW) + fo // LW
                        ocol = fo % LW
                        acc0 = av[b, pl.ds(h * L, L)] * gvec[0]
                        acc1 = av[b, pl.ds(DIM + h * L, L)] * gvec[1]
                        for k in range(2, K):
                            a16 = av[b, pl.ds(k * DIM + h * L, L)]
                            if k % 2 == 0:
                                acc0 = acc0 + a16 * gvec[k]
                            else:
                                acc1 = acc1 + a16 * gvec[k]
                        acc = acc0 + acc1
                        op[orow, pl.ds(ocol, L)] = acc
                        oe[orow, pl.ds(ocol, L)] = (
                            dv[orow, pl.ds(ocol, L)] + acc)
                return carry

            lax.fori_loop(0, C // L, obody, 0)

        def flush(c, slot):
            oe, op, sem = obufs[slot]
            oro = pl.multiple_of((base + c * C) * DIM // LW, ORC)
            return (
                pltpu.async_copy(oe, de_hbm.at[pl.ds(oro, ORC)], sem),
                pltpu.async_copy(op, dp_hbm.at[pl.ds(oro, ORC)], sem),
            )

        pend = start(0, 0)
        oflush = [None, None]
        for c in range(NCHUNK):
            slot = c & 1
            cur = pend
            if c + 1 < NCHUNK:
                pend = start(c + 1, 1 - slot)
            for cp in cur:
                cp.wait()
            if oflush[slot] is not None:
                for cp in oflush[slot]:
                    cp.wait()
            compute(c, slot)
            oflush[slot] = flush(c, slot)
        for fl in oflush:
            if fl is not None:
                for cp in fl:
                    cp.wait()

    return enc2(rel_id, A_t, dt_t, gw_t)


def kernel(rel_id, tau, W_trend, A, mu, s):
    N, K, DIM = A.shape
    B = rel_id.shape[0]
    rel32 = rel_id.astype(jnp.int32)
    A_t = A.reshape(N, K * DIM)
    st_t = jnp.concatenate(
        [W_trend, mu, s,
         jnp.zeros((N, SLOT - DIM - 2 * K), jnp.float32)],
        axis=1).reshape(N * SLOT // LW, LW)
    dt, gw = _trend_and_gauss_call(rel32, tau, st_t, B, DIM, K)
    de, dp = _pulse_call(rel32, A_t, dt, gw, B, DIM, K)
    return (de.reshape(B, DIM), dt.reshape(B, DIM), dp.reshape(B, DIM))
